# Initial kernel scaffold; baseline (speedup 1.0000x reference)
import jax
import jax.numpy as jnp
from jax.experimental import pallas as pl


def kernel(x, edge_index, W1, b1, W2, b2, W3, b3):
    raise NotImplementedError("write your pallas kernel here")


# speedup vs baseline: 5.5986x; 5.5986x over previous
import functools

import jax
import jax.numpy as jnp
from jax import lax
from jax.experimental import pallas as pl
from jax.experimental.pallas import tpu as pltpu
from jax.experimental.pallas import tpu_sc as plsc

N_NODES = 10000
NP = 10240
N_EDGES = 320000
N_IN = 128
N_HID = 128
N_CLS = 64

NS = 16
CHUNK = 128
NCHUNK = 160
EP = NS * NCHUNK * CHUNK
PAD_NODE = NP - 1
ROWS_PER_TILE = NP // NS


def _make_mesh():
    return plsc.VectorSubcoreMesh(core_axis_name="c", subcore_axis_name="s")


def _make_degrees():
    @functools.partial(
        pl.kernel,
        out_type=(
            jax.ShapeDtypeStruct((2, NP, 16), jnp.float32),
            jax.ShapeDtypeStruct((2, NP, 16), jnp.float32),
        ),
        mesh=_make_mesh(),
        compiler_params=pltpu.CompilerParams(use_tc_tiling_on_sc=False),
        scratch_types=[
            pltpu.VMEM((NCHUNK, CHUNK), jnp.int32),
            pltpu.VMEM((NCHUNK, CHUNK), jnp.int32),
            pltpu.VMEM((CHUNK, 16), jnp.float32),
            pltpu.VMEM((ROWS_PER_TILE, 16), jnp.float32),
            pltpu.VMEM_SHARED((NP, 16), jnp.float32),
        ],
    )
    def degrees(src_hbm, dst_hbm, dout_hbm, din_hbm,
                srcv, dstv, ones_v, zb, hist_sh):
        cid = lax.axis_index("c")
        sid = lax.axis_index("s")

        def fill_zero(i, carry):
            zb[i, :] = jnp.zeros((16,), jnp.float32)
            return carry
        lax.fori_loop(0, ROWS_PER_TILE, fill_zero, None)

        def fill_ones(i, carry):
            ones_v[i, :] = jnp.ones((16,), jnp.float32)
            return carry
        lax.fori_loop(0, CHUNK, fill_ones, None)

        pltpu.sync_copy(src_hbm.at[sid], srcv)
        pltpu.sync_copy(dst_hbm.at[sid], dstv)

        row0 = sid * ROWS_PER_TILE
        rows = pl.ds(row0, ROWS_PER_TILE)
        c0 = cid * (NCHUNK // 2)

        for idxv, out_hbm in ((srcv, dout_hbm), (dstv, din_hbm)):
            pltpu.sync_copy(zb, hist_sh.at[rows])
            plsc.subcore_barrier()

            def body(i, carry):
                pltpu.sync_copy(ones_v, hist_sh.at[idxv.at[c0 + i]], add=True)
                return carry
            lax.fori_loop(0, NCHUNK // 2, body, None)

            plsc.subcore_barrier()
            pltpu.sync_copy(hist_sh.at[rows], out_hbm.at[cid].at[rows])
            plsc.subcore_barrier()

    return degrees


def _make_agg(F):
    FH = F // 2

    @functools.partial(
        pl.kernel,
        out_type=jax.ShapeDtypeStruct((2, NP, FH), jnp.float32),
        mesh=_make_mesh(),
        compiler_params=pltpu.CompilerParams(use_tc_tiling_on_sc=False),
        scratch_types=[
            pltpu.VMEM((NCHUNK, CHUNK), jnp.int32),
            pltpu.VMEM((NCHUNK, CHUNK), jnp.int32),
            pltpu.VMEM((2, CHUNK, FH), jnp.float32),
            pltpu.VMEM((CHUNK, FH), jnp.float32),
            pltpu.VMEM_SHARED((NP, FH), jnp.float32),
            pltpu.SemaphoreType.DMA,
            pltpu.SemaphoreType.DMA,
        ],
    )
    def agg(h_hbm, src_hbm, dst_hbm, out_hbm,
            srcv, dstv, msg, zb, agg_sh, sem0, sem1):
        cid = lax.axis_index("c")
        sid = lax.axis_index("s")

        def fill_zero(i, carry):
            for j in range(FH // 16):
                zb[i, pl.ds(j * 16, 16)] = jnp.zeros((16,), jnp.float32)
            return carry
        lax.fori_loop(0, CHUNK, fill_zero, None)

        row0 = sid * ROWS_PER_TILE
        for k in range(ROWS_PER_TILE // CHUNK):
            pltpu.sync_copy(zb, agg_sh.at[pl.ds(row0 + k * CHUNK, CHUNK)])
        pltpu.sync_copy(src_hbm.at[sid], srcv)
        pltpu.sync_copy(dst_hbm.at[sid], dstv)
        plsc.subcore_barrier()

        h_c = h_hbm.at[cid]
        sems = [sem0, sem1]
        pltpu.async_copy(h_c.at[srcv.at[0]], msg.at[0], sem0)

        def body(g, carry):
            for b in range(2):
                cc = 2 * g + b
                nxt = cc + 1

                @pl.when(nxt < NCHUNK)
                def _issue():
                    pltpu.async_copy(h_c.at[srcv.at[nxt]],
                                     msg.at[1 - b], sems[1 - b])

                pltpu.make_async_copy(h_c.at[srcv.at[cc]],
                                      msg.at[b], sems[b]).wait()
                pltpu.sync_copy(msg.at[b], agg_sh.at[dstv.at[cc]], add=True)
            return carry
        lax.fori_loop(0, NCHUNK // 2, body, None)

        plsc.subcore_barrier()
        rows = pl.ds(row0, ROWS_PER_TILE)
        pltpu.sync_copy(agg_sh.at[rows], out_hbm.at[cid].at[rows])

    return agg


_degrees_call = _make_degrees()
_agg128_call = _make_agg(128)
_agg64_call = _make_agg(64)


_BLK = 1024
_GRID = NP // _BLK


def _norm_col(dp_ref):
    d = dp_ref[0, :, 0:1] + dp_ref[1, :, 0:1]
    return lax.rsqrt(jnp.maximum(d, 1.0))


def _deg_spec():
    return pl.BlockSpec((2, _BLK, 16), lambda j: (0, j, 0))


def _halves_spec(fh):
    return pl.BlockSpec((2, _BLK, fh), lambda j: (0, j, 0))


def _split_store(o_ref, h):
    fh = h.shape[1] // 2
    o_ref[0] = h[:, :fh]
    o_ref[1] = h[:, fh:]


def _prescale_body(x_ref, doutp_ref, o_ref):
    _split_store(o_ref, x_ref[...] * _norm_col(doutp_ref))


def _prescale(x, dout_p):
    return pl.pallas_call(
        _prescale_body,
        grid=(_GRID,),
        in_specs=[pl.BlockSpec((_BLK, N_IN), lambda j: (j, 0)), _deg_spec()],
        out_specs=_halves_spec(N_IN // 2),
        out_shape=jax.ShapeDtypeStruct((2, NP, N_IN // 2), jnp.float32),
    )(x, dout_p)


def _dense_body(ap_ref, dinp_ref, doutp_ref, w_ref, b_ref, o_ref):
    a = jnp.concatenate([ap_ref[0], ap_ref[1]], axis=1) * _norm_col(dinp_ref)
    h = jnp.dot(a, w_ref[...], preferred_element_type=jnp.float32) + b_ref[...]
    _split_store(o_ref, jnp.maximum(h, 0.0) * _norm_col(doutp_ref))


def _dense(agg_p, din_p, dout_p, w, b):
    fo = w.shape[1]
    return pl.pallas_call(
        _dense_body,
        grid=(_GRID,),
        in_specs=[
            _halves_spec(N_HID // 2),
            _deg_spec(), _deg_spec(),
            pl.BlockSpec(w.shape, lambda j: (0, 0)),
            pl.BlockSpec((1, fo), lambda j: (0, 0)),
        ],
        out_specs=_halves_spec(fo // 2),
        out_shape=jax.ShapeDtypeStruct((2, NP, fo // 2), jnp.float32),
    )(agg_p, din_p, dout_p, w, b)


def _dense2_body(ap_ref, dinp_ref, doutp_ref, w_ref, b_ref, w3_ref, o_ref):
    a = jnp.concatenate([ap_ref[0], ap_ref[1]], axis=1) * _norm_col(dinp_ref)
    h = jnp.dot(a, w_ref[...], preferred_element_type=jnp.float32) + b_ref[...]
    h = jnp.maximum(h, 0.0) * _norm_col(doutp_ref)
    _split_store(o_ref, jnp.dot(h, w3_ref[...],
                                preferred_element_type=jnp.float32))


def _dense2(agg_p, din_p, dout_p, w2, b2, w3):
    return pl.pallas_call(
        _dense2_body,
        grid=(_GRID,),
        in_specs=[
            _halves_spec(N_HID // 2),
            _deg_spec(), _deg_spec(),
            pl.BlockSpec(w2.shape, lambda j: (0, 0)),
            pl.BlockSpec((1, N_HID), lambda j: (0, 0)),
            pl.BlockSpec(w3.shape, lambda j: (0, 0)),
        ],
        out_specs=_halves_spec(N_CLS // 2),
        out_shape=jax.ShapeDtypeStruct((2, NP, N_CLS // 2), jnp.float32),
    )(agg_p, din_p, dout_p, w2, b2, w3)


def _final_body(ap_ref, dinp_ref, b_ref, o_ref):
    a = jnp.concatenate([ap_ref[0], ap_ref[1]], axis=1) * _norm_col(dinp_ref)
    o_ref[...] = a + b_ref[...]


def _final(agg_p, din_p, b3):
    return pl.pallas_call(
        _final_body,
        grid=(_GRID,),
        in_specs=[
            _halves_spec(N_CLS // 2),
            _deg_spec(),
            pl.BlockSpec((1, N_CLS), lambda j: (0, 0)),
        ],
        out_specs=pl.BlockSpec((_BLK, N_CLS), lambda j: (j, 0)),
        out_shape=jax.ShapeDtypeStruct((NP, N_CLS), jnp.float32),
    )(agg_p, din_p, b3)


def kernel(x, edge_index, W1, b1, W2, b2, W3, b3):
    src = edge_index[0].astype(jnp.int32)
    dst = edge_index[1].astype(jnp.int32)
    pad = jnp.full((EP - N_EDGES,), PAD_NODE, jnp.int32)
    srcp = jnp.concatenate([src, pad]).reshape(NS, NCHUNK, CHUNK)
    dstp = jnp.concatenate([dst, pad]).reshape(NS, NCHUNK, CHUNK)
    xp = jnp.pad(x, ((0, NP - N_NODES), (0, 0)))

    dout_p, din_p = _degrees_call(srcp, dstp)

    xs = _prescale(xp, dout_p)
    a1 = _agg128_call(xs, srcp, dstp)
    h1 = _dense(a1, din_p, dout_p, W1, b1.reshape(1, -1))
    a2 = _agg128_call(h1, srcp, dstp)
    y = _dense2(a2, din_p, dout_p, W2, b2.reshape(1, -1), W3)
    a3 = _agg64_call(y, srcp, dstp)
    out = _final(a3, din_p, b3.reshape(1, -1))
    return out[:N_NODES]


# speedup vs baseline: 5.6524x; 1.0096x over previous
import functools

import jax
import jax.numpy as jnp
from jax import lax
from jax.experimental import pallas as pl
from jax.experimental.pallas import tpu as pltpu
from jax.experimental.pallas import tpu_sc as plsc

N_NODES = 10000
NP = 10240
N_EDGES = 320000
N_IN = 128
N_HID = 128
N_CLS = 64

NS = 16
CHUNK = 128
NCHUNK = 160
EP = NS * NCHUNK * CHUNK
PAD_NODE = NP - 1
ROWS_PER_TILE = NP // NS


def _make_mesh():
    return plsc.VectorSubcoreMesh(core_axis_name="c", subcore_axis_name="s")


def _make_degrees():
    @functools.partial(
        pl.kernel,
        out_type=(
            jax.ShapeDtypeStruct((2, NP, 16), jnp.float32),
            jax.ShapeDtypeStruct((2, NP, 16), jnp.float32),
        ),
        mesh=_make_mesh(),
        compiler_params=pltpu.CompilerParams(use_tc_tiling_on_sc=False),
        scratch_types=[
            pltpu.VMEM((NCHUNK, CHUNK), jnp.int32),
            pltpu.VMEM((NCHUNK, CHUNK), jnp.int32),
            pltpu.VMEM((CHUNK, 16), jnp.float32),
            pltpu.VMEM((ROWS_PER_TILE, 16), jnp.float32),
            pltpu.VMEM_SHARED((NP, 16), jnp.float32),
        ],
    )
    def degrees(src_hbm, dst_hbm, dout_hbm, din_hbm,
                srcv, dstv, ones_v, zb, hist_sh):
        cid = lax.axis_index("c")
        sid = lax.axis_index("s")

        def fill_zero(i, carry):
            zb[i, :] = jnp.zeros((16,), jnp.float32)
            return carry
        lax.fori_loop(0, ROWS_PER_TILE, fill_zero, None)

        def fill_ones(i, carry):
            ones_v[i, :] = jnp.ones((16,), jnp.float32)
            return carry
        lax.fori_loop(0, CHUNK, fill_ones, None)

        pltpu.sync_copy(src_hbm.at[sid], srcv)
        pltpu.sync_copy(dst_hbm.at[sid], dstv)

        row0 = sid * ROWS_PER_TILE
        rows = pl.ds(row0, ROWS_PER_TILE)
        c0 = cid * (NCHUNK // 2)

        for idxv, out_hbm in ((srcv, dout_hbm), (dstv, din_hbm)):
            pltpu.sync_copy(zb, hist_sh.at[rows])
            plsc.subcore_barrier()

            def body(i, carry):
                pltpu.sync_copy(ones_v, hist_sh.at[idxv.at[c0 + i]], add=True)
                return carry
            lax.fori_loop(0, NCHUNK // 2, body, None)

            plsc.subcore_barrier()
            pltpu.sync_copy(hist_sh.at[rows], out_hbm.at[cid].at[rows])
            plsc.subcore_barrier()

    return degrees


def _make_agg(F):
    FH = F // 2

    @functools.partial(
        pl.kernel,
        out_type=jax.ShapeDtypeStruct((2, NP, FH), jnp.float32),
        mesh=_make_mesh(),
        compiler_params=pltpu.CompilerParams(use_tc_tiling_on_sc=False),
        scratch_types=[
            pltpu.VMEM((NCHUNK, CHUNK), jnp.int32),
            pltpu.VMEM((NCHUNK, CHUNK), jnp.int32),
            pltpu.VMEM((4, CHUNK, FH), jnp.float32),
            pltpu.VMEM((CHUNK, FH), jnp.float32),
            pltpu.VMEM_SHARED((NP, FH), jnp.float32),
            [pltpu.SemaphoreType.DMA] * 4,
            [pltpu.SemaphoreType.DMA] * 4,
        ],
    )
    def agg(h_hbm, src_hbm, dst_hbm, out_hbm,
            srcv, dstv, msg, zb, agg_sh, gsems, ssems):
        cid = lax.axis_index("c")
        sid = lax.axis_index("s")

        def fill_zero(i, carry):
            for j in range(FH // 16):
                zb[i, pl.ds(j * 16, 16)] = jnp.zeros((16,), jnp.float32)
            return carry
        lax.fori_loop(0, CHUNK, fill_zero, None)

        row0 = sid * ROWS_PER_TILE
        for k in range(ROWS_PER_TILE // CHUNK):
            pltpu.sync_copy(zb, agg_sh.at[pl.ds(row0 + k * CHUNK, CHUNK)])
        pltpu.sync_copy(src_hbm.at[sid], srcv)
        pltpu.sync_copy(dst_hbm.at[sid], dstv)
        plsc.subcore_barrier()

        h_c = h_hbm.at[cid]

        def gather(c, b):
            pltpu.async_copy(h_c.at[srcv.at[c]], msg.at[b], gsems[b])

        def gather_wait(c, b):
            pltpu.make_async_copy(h_c.at[srcv.at[c]], msg.at[b],
                                  gsems[b]).wait()

        def scatter(c, b):
            pltpu.async_copy(msg.at[b], agg_sh.at[dstv.at[c]], ssems[b],
                             add=True)

        def scatter_wait(c, b):
            pltpu.make_async_copy(msg.at[b], agg_sh.at[dstv.at[c]],
                                  ssems[b]).wait()

        gather(0, 0)
        gather(1, 1)

        def body(g, carry):
            for b4 in range(4):
                cc = 4 * g + b4
                gather_wait(cc, b4)
                scatter(cc, b4)
                nxt = cc + 2

                @pl.when(cc >= 2)
                def _drain():
                    scatter_wait(cc - 2, (b4 + 2) % 4)

                @pl.when(nxt < NCHUNK)
                def _refill():
                    gather(nxt, (b4 + 2) % 4)
            return carry
        lax.fori_loop(0, NCHUNK // 4, body, None)

        scatter_wait(NCHUNK - 2, (NCHUNK - 2) % 4)
        scatter_wait(NCHUNK - 1, (NCHUNK - 1) % 4)

        plsc.subcore_barrier()
        rows = pl.ds(row0, ROWS_PER_TILE)
        pltpu.sync_copy(agg_sh.at[rows], out_hbm.at[cid].at[rows])

    return agg


_degrees_call = _make_degrees()
_agg128_call = _make_agg(128)
_agg64_call = _make_agg(64)


_BLK = 1024
_GRID = NP // _BLK


def _norm_col(dp_ref):
    d = dp_ref[0, :, 0:1] + dp_ref[1, :, 0:1]
    return lax.rsqrt(jnp.maximum(d, 1.0))


def _deg_spec():
    return pl.BlockSpec((2, _BLK, 16), lambda j: (0, j, 0))


def _halves_spec(fh):
    return pl.BlockSpec((2, _BLK, fh), lambda j: (0, j, 0))


def _split_store(o_ref, h):
    fh = h.shape[1] // 2
    o_ref[0] = h[:, :fh]
    o_ref[1] = h[:, fh:]


def _prescale_body(x_ref, doutp_ref, o_ref):
    _split_store(o_ref, x_ref[...] * _norm_col(doutp_ref))


def _prescale(x, dout_p):
    return pl.pallas_call(
        _prescale_body,
        grid=(_GRID,),
        in_specs=[pl.BlockSpec((_BLK, N_IN), lambda j: (j, 0)), _deg_spec()],
        out_specs=_halves_spec(N_IN // 2),
        out_shape=jax.ShapeDtypeStruct((2, NP, N_IN // 2), jnp.float32),
    )(x, dout_p)


def _dense_body(ap_ref, dinp_ref, doutp_ref, w_ref, b_ref, o_ref):
    a = jnp.concatenate([ap_ref[0], ap_ref[1]], axis=1) * _norm_col(dinp_ref)
    h = jnp.dot(a, w_ref[...], preferred_element_type=jnp.float32) + b_ref[...]
    _split_store(o_ref, jnp.maximum(h, 0.0) * _norm_col(doutp_ref))


def _dense(agg_p, din_p, dout_p, w, b):
    fo = w.shape[1]
    return pl.pallas_call(
        _dense_body,
        grid=(_GRID,),
        in_specs=[
            _halves_spec(N_HID // 2),
            _deg_spec(), _deg_spec(),
            pl.BlockSpec(w.shape, lambda j: (0, 0)),
            pl.BlockSpec((1, fo), lambda j: (0, 0)),
        ],
        out_specs=_halves_spec(fo // 2),
        out_shape=jax.ShapeDtypeStruct((2, NP, fo // 2), jnp.float32),
    )(agg_p, din_p, dout_p, w, b)


def _dense2_body(ap_ref, dinp_ref, doutp_ref, w_ref, b_ref, w3_ref, o_ref):
    a = jnp.concatenate([ap_ref[0], ap_ref[1]], axis=1) * _norm_col(dinp_ref)
    h = jnp.dot(a, w_ref[...], preferred_element_type=jnp.float32) + b_ref[...]
    h = jnp.maximum(h, 0.0) * _norm_col(doutp_ref)
    _split_store(o_ref, jnp.dot(h, w3_ref[...],
                                preferred_element_type=jnp.float32))


def _dense2(agg_p, din_p, dout_p, w2, b2, w3):
    return pl.pallas_call(
        _dense2_body,
        grid=(_GRID,),
        in_specs=[
            _halves_spec(N_HID // 2),
            _deg_spec(), _deg_spec(),
            pl.BlockSpec(w2.shape, lambda j: (0, 0)),
            pl.BlockSpec((1, N_HID), lambda j: (0, 0)),
            pl.BlockSpec(w3.shape, lambda j: (0, 0)),
        ],
        out_specs=_halves_spec(N_CLS // 2),
        out_shape=jax.ShapeDtypeStruct((2, NP, N_CLS // 2), jnp.float32),
    )(agg_p, din_p, dout_p, w2, b2, w3)


def _final_body(ap_ref, dinp_ref, b_ref, o_ref):
    a = jnp.concatenate([ap_ref[0], ap_ref[1]], axis=1) * _norm_col(dinp_ref)
    o_ref[...] = a + b_ref[...]


def _final(agg_p, din_p, b3):
    return pl.pallas_call(
        _final_body,
        grid=(_GRID,),
        in_specs=[
            _halves_spec(N_CLS // 2),
            _deg_spec(),
            pl.BlockSpec((1, N_CLS), lambda j: (0, 0)),
        ],
        out_specs=pl.BlockSpec((_BLK, N_CLS), lambda j: (j, 0)),
        out_shape=jax.ShapeDtypeStruct((NP, N_CLS), jnp.float32),
    )(agg_p, din_p, b3)


def kernel(x, edge_index, W1, b1, W2, b2, W3, b3):
    src = edge_index[0].astype(jnp.int32)
    dst = edge_index[1].astype(jnp.int32)
    pad = jnp.full((EP - N_EDGES,), PAD_NODE, jnp.int32)
    srcp = jnp.concatenate([src, pad]).reshape(NS, NCHUNK, CHUNK)
    dstp = jnp.concatenate([dst, pad]).reshape(NS, NCHUNK, CHUNK)
    xp = jnp.pad(x, ((0, NP - N_NODES), (0, 0)))

    dout_p, din_p = _degrees_call(srcp, dstp)

    xs = _prescale(xp, dout_p)
    a1 = _agg128_call(xs, srcp, dstp)
    h1 = _dense(a1, din_p, dout_p, W1, b1.reshape(1, -1))
    a2 = _agg128_call(h1, srcp, dstp)
    y = _dense2(a2, din_p, dout_p, W2, b2.reshape(1, -1), W3)
    a3 = _agg64_call(y, srcp, dstp)
    out = _final(a3, din_p, b3.reshape(1, -1))
    return out[:N_NODES]


# speedup vs baseline: 6.5909x; 1.1660x over previous
import functools

import numpy as np

import jax
import jax.numpy as jnp
from jax import lax
from jax.experimental import pallas as pl
from jax.experimental.pallas import tpu as pltpu
from jax.experimental.pallas import tpu_sc as plsc

N_NODES = 10000
NP = 10240
N_EDGES = 320000
N_IN = 128
N_HID = 128
N_CLS = 64

NS = 16
CHUNK = 128
NCHUNK = 160
EP = NS * NCHUNK * CHUNK
PAD_NODE = NP - 1
ROWS_PER_TILE = NP // NS


def _make_mesh():
    return plsc.VectorSubcoreMesh(core_axis_name="c", subcore_axis_name="s")


def _make_degrees():
    @functools.partial(
        pl.kernel,
        out_type=(
            jax.ShapeDtypeStruct((2, NP, 16), jnp.float32),
            jax.ShapeDtypeStruct((2, NP, 16), jnp.float32),
        ),
        mesh=_make_mesh(),
        compiler_params=pltpu.CompilerParams(use_tc_tiling_on_sc=False),
        scratch_types=[
            pltpu.VMEM((NCHUNK, CHUNK), jnp.int32),
            pltpu.VMEM((NCHUNK, CHUNK), jnp.int32),
            pltpu.VMEM((CHUNK, 16), jnp.float32),
            pltpu.VMEM((ROWS_PER_TILE, 16), jnp.float32),
            pltpu.VMEM_SHARED((NP, 16), jnp.float32),
        ],
    )
    def degrees(src_hbm, dst_hbm, dout_hbm, din_hbm,
                srcv, dstv, ones_v, zb, hist_sh):
        cid = lax.axis_index("c")
        sid = lax.axis_index("s")

        def fill_zero(i, carry):
            zb[i, :] = jnp.zeros((16,), jnp.float32)
            return carry
        lax.fori_loop(0, ROWS_PER_TILE, fill_zero, None)

        def fill_ones(i, carry):
            ones_v[i, :] = jnp.ones((16,), jnp.float32)
            return carry
        lax.fori_loop(0, CHUNK, fill_ones, None)

        pltpu.sync_copy(src_hbm.at[sid], srcv)
        pltpu.sync_copy(dst_hbm.at[sid], dstv)

        row0 = sid * ROWS_PER_TILE
        rows = pl.ds(row0, ROWS_PER_TILE)
        c0 = cid * (NCHUNK // 2)

        for idxv, out_hbm in ((srcv, dout_hbm), (dstv, din_hbm)):
            pltpu.sync_copy(zb, hist_sh.at[rows])
            plsc.subcore_barrier()

            def body(i, carry):
                pltpu.sync_copy(ones_v, hist_sh.at[idxv.at[c0 + i]], add=True)
                return carry
            lax.fori_loop(0, NCHUNK // 2, body, None)

            plsc.subcore_barrier()
            pltpu.sync_copy(hist_sh.at[rows], out_hbm.at[cid].at[rows])
            plsc.subcore_barrier()

    return degrees


def _make_agg(F):
    FH = F // 2

    @functools.partial(
        pl.kernel,
        out_type=jax.ShapeDtypeStruct((2, NP, FH), jnp.float32),
        mesh=_make_mesh(),
        compiler_params=pltpu.CompilerParams(use_tc_tiling_on_sc=False,
                                             needs_layout_passes=False),
        scratch_types=[
            pltpu.VMEM((NCHUNK, CHUNK), jnp.int32),
            pltpu.VMEM((NCHUNK, CHUNK), jnp.int32),
            pltpu.VMEM((4, CHUNK, FH // 2), jnp.int32),
            pltpu.VMEM((2, CHUNK, FH), jnp.float32),
            pltpu.VMEM((CHUNK, FH), jnp.float32),
            pltpu.VMEM_SHARED((NP, FH), jnp.float32),
            [pltpu.SemaphoreType.DMA] * 4,
            [pltpu.SemaphoreType.DMA] * 2,
        ],
    )
    def agg(h_hbm, src_hbm, dst_hbm, out_hbm,
            srcv, dstv, bmsg, fmsg, zb, agg_sh, gsems, ssems):
        cid = lax.axis_index("c")
        sid = lax.axis_index("s")

        def fill_zero(i, carry):
            for j in range(FH // 16):
                zb[i, pl.ds(j * 16, 16)] = jnp.zeros((16,), jnp.float32)
            return carry
        lax.fori_loop(0, CHUNK, fill_zero, None)

        row0 = sid * ROWS_PER_TILE
        for k in range(ROWS_PER_TILE // CHUNK):
            pltpu.sync_copy(zb, agg_sh.at[pl.ds(row0 + k * CHUNK, CHUNK)])
        pltpu.sync_copy(src_hbm.at[sid], srcv)
        pltpu.sync_copy(dst_hbm.at[sid], dstv)
        plsc.subcore_barrier()

        h_c = h_hbm.at[cid]

        def gather(c, b):
            pltpu.async_copy(h_c.at[srcv.at[c]], bmsg.at[b], gsems[b])

        def gather_wait(c, b):
            pltpu.make_async_copy(h_c.at[srcv.at[c]], bmsg.at[b],
                                  gsems[b]).wait()

        def scatter(c, f):
            pltpu.async_copy(fmsg.at[f], agg_sh.at[dstv.at[c]], ssems[f],
                             add=True)

        def scatter_wait(c, f):
            pltpu.make_async_copy(fmsg.at[f], agg_sh.at[dstv.at[c]],
                                  ssems[f]).wait()

        def convert(b, f):
            def conv_row(r, carry):
                for j in range(FH // 32):
                    u = bmsg[b, r, pl.ds(16 * j, 16)]
                    lo = plsc.bitcast(u << 16, jnp.float32)
                    hi = plsc.bitcast(u & jnp.int32(-65536), jnp.float32)
                    fmsg[f, r, pl.ds(32 * j, 16)] = lo
                    fmsg[f, r, pl.ds(32 * j + 16, 16)] = hi
                return carry
            lax.fori_loop(0, CHUNK, conv_row, None)

        gather(0, 0)
        gather(1, 1)

        def body(g, carry):
            for b4 in range(4):
                cc = 4 * g + b4
                f2 = b4 % 2

                @pl.when(cc >= 2)
                def _drain():
                    scatter_wait(cc - 2, f2)

                gather_wait(cc, b4)

                @pl.when(cc + 2 < NCHUNK)
                def _refill():
                    gather(cc + 2, (b4 + 2) % 4)

                convert(b4, f2)
                scatter(cc, f2)
            return carry
        lax.fori_loop(0, NCHUNK // 4, body, None)

        scatter_wait(NCHUNK - 2, 0)
        scatter_wait(NCHUNK - 1, 1)

        plsc.subcore_barrier()
        rows = pl.ds(row0, ROWS_PER_TILE)
        pltpu.sync_copy(agg_sh.at[rows], out_hbm.at[cid].at[rows])

    return agg


_degrees_call = _make_degrees()
_agg128_call = _make_agg(128)
_agg64_call = _make_agg(64)


_BLK = 1024
_GRID = NP // _BLK


def _norm_col(dp_ref):
    d = dp_ref[0, :, 0:1] + dp_ref[1, :, 0:1]
    return lax.rsqrt(jnp.maximum(d, 1.0))


def _deg_spec():
    return pl.BlockSpec((2, _BLK, 16), lambda j: (0, j, 0))


def _halves_spec(fh):
    return pl.BlockSpec((2, _BLK, fh), lambda j: (0, j, 0))


def _split_store(o_ref, h):
    fh = h.shape[1] // 2
    o_ref[0] = h[:, :fh].astype(jnp.bfloat16)
    o_ref[1] = h[:, fh:].astype(jnp.bfloat16)


_SIGINV32 = np.empty(32, np.int32)
for _k in range(16):
    _SIGINV32[2 * _k] = _k
    _SIGINV32[2 * _k + 1] = 16 + _k


def _colperm(w):
    return np.concatenate([32 * g + _SIGINV32 for g in range(w // 32)])


_P128 = _colperm(128)
_P64 = _colperm(64)


def _prescale_body(x_ref, doutp_ref, o_ref):
    _split_store(o_ref, x_ref[...] * _norm_col(doutp_ref))


def _prescale(x, dout_p):
    return pl.pallas_call(
        _prescale_body,
        grid=(_GRID,),
        in_specs=[pl.BlockSpec((_BLK, N_IN), lambda j: (j, 0)), _deg_spec()],
        out_specs=_halves_spec(N_IN // 2),
        out_shape=jax.ShapeDtypeStruct((2, NP, N_IN // 2), jnp.bfloat16),
    )(x, dout_p)


def _dense_body(ap_ref, dinp_ref, doutp_ref, w_ref, b_ref, o_ref):
    a = jnp.concatenate([ap_ref[0], ap_ref[1]], axis=1) * _norm_col(dinp_ref)
    h = jnp.dot(a, w_ref[...], preferred_element_type=jnp.float32) + b_ref[...]
    _split_store(o_ref, jnp.maximum(h, 0.0) * _norm_col(doutp_ref))


def _dense(agg_p, din_p, dout_p, w, b):
    fo = w.shape[1]
    return pl.pallas_call(
        _dense_body,
        grid=(_GRID,),
        in_specs=[
            _halves_spec(N_HID // 2),
            _deg_spec(), _deg_spec(),
            pl.BlockSpec(w.shape, lambda j: (0, 0)),
            pl.BlockSpec((1, fo), lambda j: (0, 0)),
        ],
        out_specs=_halves_spec(fo // 2),
        out_shape=jax.ShapeDtypeStruct((2, NP, fo // 2), jnp.bfloat16),
    )(agg_p, din_p, dout_p, w, b)


def _dense2_body(ap_ref, dinp_ref, doutp_ref, w_ref, b_ref, w3_ref, o_ref):
    a = jnp.concatenate([ap_ref[0], ap_ref[1]], axis=1) * _norm_col(dinp_ref)
    h = jnp.dot(a, w_ref[...], preferred_element_type=jnp.float32) + b_ref[...]
    h = jnp.maximum(h, 0.0) * _norm_col(doutp_ref)
    _split_store(o_ref, jnp.dot(h, w3_ref[...],
                                preferred_element_type=jnp.float32))


def _dense2(agg_p, din_p, dout_p, w2, b2, w3):
    return pl.pallas_call(
        _dense2_body,
        grid=(_GRID,),
        in_specs=[
            _halves_spec(N_HID // 2),
            _deg_spec(), _deg_spec(),
            pl.BlockSpec(w2.shape, lambda j: (0, 0)),
            pl.BlockSpec((1, N_HID), lambda j: (0, 0)),
            pl.BlockSpec(w3.shape, lambda j: (0, 0)),
        ],
        out_specs=_halves_spec(N_CLS // 2),
        out_shape=jax.ShapeDtypeStruct((2, NP, N_CLS // 2), jnp.bfloat16),
    )(agg_p, din_p, dout_p, w2, b2, w3)


def _final_body(ap_ref, dinp_ref, b_ref, o_ref):
    a = jnp.concatenate([ap_ref[0], ap_ref[1]], axis=1) * _norm_col(dinp_ref)
    o_ref[...] = a + b_ref[...]


def _final(agg_p, din_p, b3):
    return pl.pallas_call(
        _final_body,
        grid=(_GRID,),
        in_specs=[
            _halves_spec(N_CLS // 2),
            _deg_spec(),
            pl.BlockSpec((1, N_CLS), lambda j: (0, 0)),
        ],
        out_specs=pl.BlockSpec((_BLK, N_CLS), lambda j: (j, 0)),
        out_shape=jax.ShapeDtypeStruct((NP, N_CLS), jnp.float32),
    )(agg_p, din_p, b3)


def _pack_i32(a):
    return jax.lax.bitcast_convert_type(
        a.reshape(2, NP, -1, 2), jnp.int32)


def kernel(x, edge_index, W1, b1, W2, b2, W3, b3):
    src = edge_index[0].astype(jnp.int32)
    dst = edge_index[1].astype(jnp.int32)
    pad = jnp.full((EP - N_EDGES,), PAD_NODE, jnp.int32)
    srcp = jnp.concatenate([src, pad]).reshape(NS, NCHUNK, CHUNK)
    dstp = jnp.concatenate([dst, pad]).reshape(NS, NCHUNK, CHUNK)
    xp = jnp.pad(x, ((0, NP - N_NODES), (0, 0)))[:, _P128]

    dout_p, din_p = _degrees_call(srcp, dstp)

    xs = _pack_i32(_prescale(xp, dout_p))
    a1 = _agg128_call(xs, srcp, dstp)
    h1 = _pack_i32(_dense(a1, din_p, dout_p, W1[:, _P128],
                          b1[_P128].reshape(1, -1)))
    a2 = _agg128_call(h1, srcp, dstp)
    y = _pack_i32(_dense2(a2, din_p, dout_p, W2[:, _P128],
                          b2[_P128].reshape(1, -1), W3[_P128][:, _P64]))
    a3 = _agg64_call(y, srcp, dstp)
    out = _final(a3, din_p, b3.reshape(1, -1))
    return out[:N_NODES]
